# baseline (speedup 1.0000x reference)
import functools

import jax
import jax.numpy as jnp
from jax.experimental import pallas as pl
from jax.experimental.pallas import tpu as pltpu


def _coupling_kernel(x_ref, w1_ref, b1_ref, w2_ref, b2_ref, w3_ref, b3_ref,
                     z_ref, logdet_ref, w1p, w3p, bp, *, split, out):
    H = w2_ref.shape[0]

    @pl.when(pl.program_id(0) == 0)
    def _prep():
        w1p[...] = jnp.concatenate(
            [w1_ref[...], jnp.zeros((split, H), jnp.float32)], axis=0)
        zeros_w = jnp.zeros((H, split), jnp.float32)
        w3p[...] = jnp.concatenate(
            [zeros_w, w3_ref[:, :out], zeros_w, w3_ref[:, out:]], axis=1)
        zeros_b = jnp.zeros((1, split), jnp.float32)
        brow = jnp.concatenate(
            [zeros_b, b3_ref[:, :out], zeros_b, b3_ref[:, out:]], axis=1)
        bp[...] = jnp.broadcast_to(brow, bp.shape)

    x = x_ref[...]
    h = jnp.dot(x, w1p[...], preferred_element_type=jnp.float32)
    h = jnp.maximum(h + b1_ref[...], 0.0)
    h = jnp.dot(h, w2_ref[...], preferred_element_type=jnp.float32)
    h = jnp.maximum(h + b2_ref[...], 0.0)
    p = jnp.dot(h, w3p[...], preferred_element_type=jnp.float32)
    p = p + bp[0:1, :]

    pa = p[:, :2 * split]
    ps = p[:, 2 * split:]

    z_ref[...] = x * jnp.exp(pa) + ps
    ld = jnp.sum(pa, axis=1)
    logdet_ref[...] = ld.reshape(logdet_ref.shape)


def kernel(x, w1, b1, w2, b2, w3, b3, *, tb=2048):
    B, D = x.shape
    split = D // 2
    out = D - split
    H = w1.shape[1]

    tb = max(128, min(B, tb))
    grid = (pl.cdiv(B, tb),)

    body = functools.partial(_coupling_kernel, split=split, out=out)

    def const_spec(shape):
        return pl.BlockSpec(shape, lambda i: (0,) * len(shape))

    z, logdet = pl.pallas_call(
        body,
        out_shape=(jax.ShapeDtypeStruct((B, D), x.dtype),
                   jax.ShapeDtypeStruct((B // 128, 128), jnp.float32)),
        grid=grid,
        in_specs=[
            pl.BlockSpec((tb, D), lambda i: (i, 0)),
            const_spec((split, H)),
            const_spec((1, H)),
            const_spec((H, H)),
            const_spec((1, H)),
            const_spec((H, 2 * out)),
            const_spec((1, 2 * out)),
        ],
        out_specs=(pl.BlockSpec((tb, D), lambda i: (i, 0)),
                   pl.BlockSpec((tb // 128, 128), lambda i: (i, 0))),
        scratch_shapes=[
            pltpu.VMEM((2 * split, H), jnp.float32),
            pltpu.VMEM((H, 4 * out), jnp.float32),
            pltpu.VMEM((8, 4 * out), jnp.float32),
        ],
        compiler_params=pltpu.CompilerParams(
            dimension_semantics=("arbitrary",)),
    )(x, w1, b1.reshape(1, -1), w2, b2.reshape(1, -1),
      w3, b3.reshape(1, -1))

    return z, logdet.reshape(B)


# speedup vs baseline: 1.4054x; 1.4054x over previous
import functools

import jax
import jax.numpy as jnp
from jax.experimental import pallas as pl
from jax.experimental.pallas import tpu as pltpu


def _coupling_kernel(x_ref, w1_ref, b1_ref, w2_ref, b2_ref, w3_ref, b3_ref,
                     z_ref, logdet_ref, w1p, w3p, bp, *, split, out):
    H = w2_ref.shape[0]

    @pl.when(pl.program_id(0) == 0)
    def _prep():
        w1p[...] = jnp.concatenate(
            [w1_ref[...], jnp.zeros((split, H), jnp.float32)], axis=0)
        zeros_w = jnp.zeros((H, split), jnp.float32)
        w3p[...] = jnp.concatenate(
            [zeros_w, w3_ref[:, :out], zeros_w, w3_ref[:, out:]], axis=1)
        zeros_b = jnp.zeros((1, split), jnp.float32)
        brow = jnp.concatenate(
            [zeros_b, b3_ref[:, :out], zeros_b, b3_ref[:, out:]], axis=1)
        bp[...] = jnp.broadcast_to(brow, bp.shape)

    x = x_ref[...]
    h = jnp.dot(x, w1p[...], preferred_element_type=jnp.float32)
    h = jnp.maximum(h + b1_ref[...], 0.0)
    h = jnp.dot(h, w2_ref[...], preferred_element_type=jnp.float32)
    h = jnp.maximum(h + b2_ref[...], 0.0)
    p = jnp.dot(h, w3p[...], preferred_element_type=jnp.float32)
    p = p + bp[0:1, :]

    pa = p[:, :2 * split]
    ps = p[:, 2 * split:]

    z_ref[...] = x * jnp.exp(pa) + ps
    ld = jnp.sum(pa, axis=1)
    logdet_ref[...] = ld.reshape(logdet_ref.shape)


def kernel(x, w1, b1, w2, b2, w3, b3, *, tb=8192):
    B, D = x.shape
    split = D // 2
    out = D - split
    H = w1.shape[1]

    tb = max(128, min(B, tb))
    grid = (pl.cdiv(B, tb),)

    body = functools.partial(_coupling_kernel, split=split, out=out)

    def const_spec(shape):
        return pl.BlockSpec(shape, lambda i: (0,) * len(shape))

    z, logdet = pl.pallas_call(
        body,
        out_shape=(jax.ShapeDtypeStruct((B, D), x.dtype),
                   jax.ShapeDtypeStruct((B // 128, 128), jnp.float32)),
        grid=grid,
        in_specs=[
            pl.BlockSpec((tb, D), lambda i: (i, 0)),
            const_spec((split, H)),
            const_spec((1, H)),
            const_spec((H, H)),
            const_spec((1, H)),
            const_spec((H, 2 * out)),
            const_spec((1, 2 * out)),
        ],
        out_specs=(pl.BlockSpec((tb, D), lambda i: (i, 0)),
                   pl.BlockSpec((tb // 128, 128), lambda i: (i, 0))),
        scratch_shapes=[
            pltpu.VMEM((2 * split, H), jnp.float32),
            pltpu.VMEM((H, 4 * out), jnp.float32),
            pltpu.VMEM((8, 4 * out), jnp.float32),
        ],
        compiler_params=pltpu.CompilerParams(
            dimension_semantics=("arbitrary",)),
    )(x, w1, b1.reshape(1, -1), w2, b2.reshape(1, -1),
      w3, b3.reshape(1, -1))

    return z, logdet.reshape(B)
